# baseline (speedup 1.0000x reference)
import dataclasses

import jax
import jax.numpy as jnp
from jax.experimental import pallas as pl
from jax.experimental.pallas import tpu as pltpu
from jax.experimental.pallas import tpu_sc as plsc

NUM_NODES = 16384
NUM_NEIGHBORS = 50
EMBED_DIM = 64

_GATHER_WINDOW = 128
_BLOCK_PAIRS = 128
_NUM_CHUNKS = 4
_PREP_BLOCK = 10000


def _sc_gather_packed(table, idx):
    total = idx.shape[0]
    W = _GATHER_WINDOW
    mesh = plsc.VectorSubcoreMesh(core_axis_name="core", subcore_axis_name="subcore")
    idx2 = idx.reshape(1, total)

    cp = pltpu.CompilerParams()
    if "needs_layout_passes" in pltpu.CompilerParams.__dataclass_fields__:
        cp = dataclasses.replace(cp, needs_layout_passes=False)

    @pl.kernel(
        out_type=jax.ShapeDtypeStruct((total // 2, 128), table.dtype),
        mesh=mesh,
        compiler_params=cp,
        scratch_types=[
            pltpu.VMEM((W, 128), table.dtype),
            pltpu.VMEM((2, W), jnp.int32),
            pltpu.SemaphoreType.DMA((2,)),
        ],
    )
    def gather_kernel(x_hbm, i_hbm, o_hbm, tmp, parq, sems):
        H = W // 2

        def body(i_vmem, o_vmem):
            @pl.loop(0, W, step=16)
            def _(c):
                ivv = i_vmem[0, pl.ds(c, 16)]
                t = ivv // _PREP_BLOCK
                r = ivv - t * _PREP_BLOCK
                hi = (r >= (_PREP_BLOCK // 2)).astype(jnp.int32)
                parq[1, pl.ds(c, 16)] = (
                    t * (_PREP_BLOCK // 2) + r - hi * (_PREP_BLOCK // 2))
                parq[0, pl.ds(c, 16)] = hi * 64

            cp0 = pltpu.make_async_copy(
                x_hbm.at[parq.at[1, pl.ds(0, H)]], tmp.at[pl.ds(0, H)],
                sems.at[0])
            cp0.start()
            cp1 = pltpu.make_async_copy(
                x_hbm.at[parq.at[1, pl.ds(H, H)]], tmp.at[pl.ds(H, H)],
                sems.at[1])
            cp1.start()
            cp0.wait()

            def select(base):
                @pl.loop(base, base + H, step=16)
                def _(g0):
                    pvv = parq[0, pl.ds(g0, 16)]
                    for j in range(16):
                        p = pvv[j]
                        q = (g0 >> 1) + (j >> 1)
                        hb = (j & 1) * 64
                        for c in (0, 16, 32, 48):
                            o_vmem[q, pl.ds(hb + c, 16)] = (
                                tmp[g0 + j, pl.ds(p + c, 16)])

            select(0)
            cp1.wait()
            select(H)

        pltpu.emit_pipeline(
            body,
            grid=(total // W,),
            in_specs=[pl.BlockSpec((1, W), index_map=lambda i: (0, i))],
            out_specs=[pl.BlockSpec((W // 2, 128), index_map=lambda i: (i, 0))],
            core_axis_name=("core", "subcore"),
            dimension_semantics=(pltpu.PARALLEL,),
        )(i_hbm, o_hbm)

    return gather_kernel(table, idx2)


def _prep_body(x_ref, o_ref):
    x = x_ref[...]
    H = x.shape[0] // 2
    o_ref[...] = jnp.concatenate([x[:H], x[H:]], axis=1)


def _prep_table(table):
    n = table.shape[0]
    P = _PREP_BLOCK
    return pl.pallas_call(
        _prep_body,
        grid=(n // P,),
        in_specs=[pl.BlockSpec((P, 64), lambda i: (i, 0))],
        out_specs=pl.BlockSpec((P // 2, 128), lambda i: (i, 0)),
        out_shape=jax.ShapeDtypeStruct((n // 2, 128), jnp.float32),
    )(table)


def _mlp_body(g_ref, w1e_ref, w1u_ref, b1_ref, w2_ref, b2_ref, w3p_ref,
              ex_ref, b3_ref, wls_ref, wln_ref, bl_ref, o_ref):
    K = NUM_NEIGHBORS
    g = g_ref[...]
    u = g[K]
    e3 = g[:K]
    Bp = u.shape[0]

    h0 = jnp.dot(u, w1u_ref[...], preferred_element_type=jnp.float32) + b1_ref[...]
    a = jnp.dot(e3.reshape(K * Bp, 128), w1e_ref[...],
                preferred_element_type=jnp.float32)
    h1 = jax.nn.relu(a.reshape(K, Bp, 128) + h0[None])

    h2 = jax.nn.relu(
        jnp.dot(h1.reshape(K * Bp, 128), w2_ref[...],
                preferred_element_type=jnp.float32) + b2_ref[...]
    )

    z = jnp.dot(h2, w3p_ref[...], preferred_element_type=jnp.float32) + b3_ref[0, 0]
    z3 = z.reshape(K, Bp, 2)
    m = jnp.max(z3, axis=0, keepdims=True)
    p = jnp.exp(z3 - m)
    att = p / jnp.sum(p, axis=0, keepdims=True)

    att128 = jnp.dot(att.reshape(K * Bp, 2), ex_ref[...],
                     preferred_element_type=jnp.float32)
    neigh = jnp.sum(e3 * att128.reshape(K, Bp, 128), axis=0)

    out = jax.nn.relu(
        jnp.dot(u, wls_ref[...], preferred_element_type=jnp.float32)
        + jnp.dot(neigh, wln_ref[...], preferred_element_type=jnp.float32)
        + bl_ref[...]
    )
    o_ref[...] = out


def _tc_mlp(g, w1e, w1u, b1, w2, b2, w3p, ex, b3, wls, wln, bl):
    rows = g.shape[1]
    Bp = _BLOCK_PAIRS
    full = lambda shape: pl.BlockSpec(shape, lambda i: (0,) * len(shape))
    return pl.pallas_call(
        _mlp_body,
        grid=(rows // Bp,),
        in_specs=[
            pl.BlockSpec((NUM_NEIGHBORS + 1, Bp, 128), lambda i: (0, i, 0)),
            full(w1e.shape), full(w1u.shape), full(b1.shape),
            full(w2.shape), full(b2.shape), full(w3p.shape), full(ex.shape),
            full(b3.shape), full(wls.shape), full(wln.shape), full(bl.shape),
        ],
        out_specs=pl.BlockSpec((Bp, 128), lambda i: (i, 0)),
        out_shape=jax.ShapeDtypeStruct((rows, 128), jnp.float32),
    )(g, w1e, w1u, b1, w2, b2, w3p, ex, b3, wls, wln, bl)


def kernel(nodes, neighbors, u2e_weight, att1_W, att1_b, att2_W, att2_b,
           att3_W, att3_b, lin1_W, lin1_b):
    D = EMBED_DIM
    K = NUM_NEIGHBORS
    neighT = neighbors.T

    eye2 = jnp.eye(2, dtype=jnp.float32)
    w1e = jnp.kron(eye2, att1_W[:, :D].T)
    w1u = jnp.kron(eye2, att1_W[:, D:].T)
    b1 = jnp.tile(att1_b.reshape(1, D), (1, 2))
    w2 = jnp.kron(eye2, att2_W.T)
    b2 = jnp.tile(att2_b.reshape(1, D), (1, 2))
    w3p = jnp.kron(eye2, att3_W.T)
    ex = jnp.kron(eye2, jnp.ones((1, D), jnp.float32))
    b3 = att3_b.reshape(1, 1)
    wls = jnp.kron(eye2, lin1_W[:, :D].T)
    wln = jnp.kron(eye2, lin1_W[:, D:].T)
    bl = jnp.tile(lin1_b.reshape(1, D), (1, 2))

    table2 = _prep_table(u2e_weight)
    n_chunk = NUM_NODES // _NUM_CHUNKS
    outs = []
    for c in range(_NUM_CHUNKS):
        sl = slice(c * n_chunk, (c + 1) * n_chunk)
        idx = jnp.concatenate(
            [neighT[:, sl].reshape(-1), nodes[sl]]).astype(jnp.int32)
        packed = _sc_gather_packed(table2, idx)
        g = packed.reshape(K + 1, n_chunk // 2, 128)
        outs.append(_tc_mlp(g, w1e, w1u, b1, w2, b2, w3p, ex, b3,
                            wls, wln, bl))
    return jnp.concatenate(outs, axis=0).reshape(NUM_NODES, D)


# speedup vs baseline: 1.2043x; 1.2043x over previous
import dataclasses

import jax
import jax.numpy as jnp
from jax.experimental import pallas as pl
from jax.experimental.pallas import tpu as pltpu
from jax.experimental.pallas import tpu_sc as plsc

NUM_NODES = 16384
NUM_NEIGHBORS = 50
EMBED_DIM = 64

_GATHER_WINDOW = 256
_BLOCK_PAIRS = 128
_NUM_CHUNKS = 4
_PREP_BLOCK = 10000


def _sc_gather_packed(table, idx):
    total = idx.shape[0]
    W = _GATHER_WINDOW
    mesh = plsc.VectorSubcoreMesh(core_axis_name="core", subcore_axis_name="subcore")
    idx2 = idx.reshape(1, total)

    cp = pltpu.CompilerParams()
    if "needs_layout_passes" in pltpu.CompilerParams.__dataclass_fields__:
        cp = dataclasses.replace(cp, needs_layout_passes=False)

    @pl.kernel(
        out_type=jax.ShapeDtypeStruct((total // 2, 128), table.dtype),
        mesh=mesh,
        compiler_params=cp,
        scratch_types=[
            pltpu.VMEM((W, 128), table.dtype),
            pltpu.VMEM((2, W), jnp.int32),
            pltpu.SemaphoreType.DMA((2,)),
        ],
    )
    def gather_kernel(x_hbm, i_hbm, o_hbm, tmp, parq, sems):
        H = W // 2

        def body(i_vmem, o_vmem):
            @pl.loop(0, W, step=16)
            def _(c):
                ivv = i_vmem[0, pl.ds(c, 16)]
                parq[1, pl.ds(c, 16)] = ivv >> 1
                parq[0, pl.ds(c, 16)] = (ivv & 1) * 64

            cp0 = pltpu.make_async_copy(
                x_hbm.at[parq.at[1, pl.ds(0, H)]], tmp.at[pl.ds(0, H)],
                sems.at[0])
            cp0.start()
            cp1 = pltpu.make_async_copy(
                x_hbm.at[parq.at[1, pl.ds(H, H)]], tmp.at[pl.ds(H, H)],
                sems.at[1])
            cp1.start()
            cp0.wait()

            def select(base):
                @pl.loop(base, base + H, step=16)
                def _(g0):
                    pvv = parq[0, pl.ds(g0, 16)]
                    for j in range(16):
                        p = pvv[j]
                        q = (g0 >> 1) + (j >> 1)
                        hb = (j & 1) * 64
                        for c in (0, 16, 32, 48):
                            o_vmem[q, pl.ds(hb + c, 16)] = (
                                tmp[g0 + j, pl.ds(p + c, 16)])

            select(0)
            cp1.wait()
            select(H)

        pltpu.emit_pipeline(
            body,
            grid=(total // W,),
            in_specs=[pl.BlockSpec((1, W), index_map=lambda i: (0, i))],
            out_specs=[pl.BlockSpec((W // 2, 128), index_map=lambda i: (i, 0))],
            core_axis_name=("core", "subcore"),
            dimension_semantics=(pltpu.PARALLEL,),
        )(i_hbm, o_hbm)

    return gather_kernel(table, idx2)


def _prep_body(x_ref, o_ref):
    x = x_ref[...]
    H = x.shape[0] // 2
    o_ref[...] = jnp.concatenate([x[:H], x[H:]], axis=1)


def _prep_table(table):
    n = table.shape[0]
    P = _PREP_BLOCK
    return pl.pallas_call(
        _prep_body,
        grid=(n // P,),
        in_specs=[pl.BlockSpec((P, 64), lambda i: (i, 0))],
        out_specs=pl.BlockSpec((P // 2, 128), lambda i: (i, 0)),
        out_shape=jax.ShapeDtypeStruct((n // 2, 128), jnp.float32),
    )(table)


def _mlp_body(g_ref, w1e_ref, w1u_ref, b1_ref, w2_ref, b2_ref, w3p_ref,
              ex_ref, b3_ref, wls_ref, wln_ref, bl_ref, o_ref):
    K = NUM_NEIGHBORS
    g = g_ref[...]
    u = g[K]
    e3 = g[:K]
    Bp = u.shape[0]

    h0 = jnp.dot(u, w1u_ref[...], preferred_element_type=jnp.float32) + b1_ref[...]
    a = jnp.dot(e3.reshape(K * Bp, 128), w1e_ref[...],
                preferred_element_type=jnp.float32)
    h1 = jax.nn.relu(a.reshape(K, Bp, 128) + h0[None])

    h2 = jax.nn.relu(
        jnp.dot(h1.reshape(K * Bp, 128), w2_ref[...],
                preferred_element_type=jnp.float32) + b2_ref[...]
    )

    z = jnp.dot(h2, w3p_ref[...], preferred_element_type=jnp.float32) + b3_ref[0, 0]
    z3 = z.reshape(K, Bp, 2)
    m = jnp.max(z3, axis=0, keepdims=True)
    p = jnp.exp(z3 - m)
    att = p / jnp.sum(p, axis=0, keepdims=True)

    att128 = jnp.dot(att.reshape(K * Bp, 2), ex_ref[...],
                     preferred_element_type=jnp.float32)
    neigh = jnp.sum(e3 * att128.reshape(K, Bp, 128), axis=0)

    out = jax.nn.relu(
        jnp.dot(u, wls_ref[...], preferred_element_type=jnp.float32)
        + jnp.dot(neigh, wln_ref[...], preferred_element_type=jnp.float32)
        + bl_ref[...]
    )
    o_ref[...] = out


def _tc_mlp(g, w1e, w1u, b1, w2, b2, w3p, ex, b3, wls, wln, bl):
    rows = g.shape[1]
    Bp = _BLOCK_PAIRS
    full = lambda shape: pl.BlockSpec(shape, lambda i: (0,) * len(shape))
    return pl.pallas_call(
        _mlp_body,
        grid=(rows // Bp,),
        in_specs=[
            pl.BlockSpec((NUM_NEIGHBORS + 1, Bp, 128), lambda i: (0, i, 0)),
            full(w1e.shape), full(w1u.shape), full(b1.shape),
            full(w2.shape), full(b2.shape), full(w3p.shape), full(ex.shape),
            full(b3.shape), full(wls.shape), full(wln.shape), full(bl.shape),
        ],
        out_specs=pl.BlockSpec((Bp, 128), lambda i: (i, 0)),
        out_shape=jax.ShapeDtypeStruct((rows, 128), jnp.float32),
    )(g, w1e, w1u, b1, w2, b2, w3p, ex, b3, wls, wln, bl)


def kernel(nodes, neighbors, u2e_weight, att1_W, att1_b, att2_W, att2_b,
           att3_W, att3_b, lin1_W, lin1_b):
    D = EMBED_DIM
    K = NUM_NEIGHBORS
    neighT = neighbors.T

    eye2 = jnp.eye(2, dtype=jnp.float32)
    w1e = jnp.kron(eye2, att1_W[:, :D].T)
    w1u = jnp.kron(eye2, att1_W[:, D:].T)
    b1 = jnp.tile(att1_b.reshape(1, D), (1, 2))
    w2 = jnp.kron(eye2, att2_W.T)
    b2 = jnp.tile(att2_b.reshape(1, D), (1, 2))
    w3p = jnp.kron(eye2, att3_W.T)
    ex = jnp.kron(eye2, jnp.ones((1, D), jnp.float32))
    b3 = att3_b.reshape(1, 1)
    wls = jnp.kron(eye2, lin1_W[:, :D].T)
    wln = jnp.kron(eye2, lin1_W[:, D:].T)
    bl = jnp.tile(lin1_b.reshape(1, D), (1, 2))

    table2 = u2e_weight.reshape(-1, 2 * D)
    n_chunk = NUM_NODES // _NUM_CHUNKS
    outs = []
    for c in range(_NUM_CHUNKS):
        sl = slice(c * n_chunk, (c + 1) * n_chunk)
        idx = jnp.concatenate(
            [neighT[:, sl].reshape(-1), nodes[sl]]).astype(jnp.int32)
        packed = _sc_gather_packed(table2, idx)
        g = packed.reshape(K + 1, n_chunk // 2, 128)
        outs.append(_tc_mlp(g, w1e, w1u, b1, w2, b2, w3p, ex, b3,
                            wls, wln, bl))
    return jnp.concatenate(outs, axis=0).reshape(NUM_NODES, D)
